# baseline (speedup 1.0000x reference)
import functools

import jax
import jax.numpy as jnp
from jax import lax
from jax.experimental import pallas as pl
from jax.experimental.pallas import tpu as pltpu
from jax.experimental.pallas import tpu_sc as plsc

N = 50000
E = 800000
D = 64
G = 512
NC = 2
NS = 16
NW = NC * NS
L = 16

QSTEP = 12544
QS_LAST = N - 3 * QSTEP
DUMMY = QSTEP
ACC_R = 12672
WB = QSTEP // NS
WB3 = 776
NPAD = 3 * QSTEP + NS * WB3

EW = E // NW
RS = (6272, 6272, 6272, 6184)
RO = (0, 6272, 12544, 18816)
CAPR = 6400
NR = 4
CHUNK = 128

DEG_EW = 50048
DEG_BLK = 23
DEG_NB = 17
HALF = 2 * QSTEP
DEG_ACC = HALF + 128

FMIN = -3.0e38

_mesh = plsc.VectorSubcoreMesh(core_axis_name="c", subcore_axis_name="s",
                               num_cores=NC, num_subcores=NS)
_sc_params = pltpu.CompilerParams(needs_layout_passes=False)


def _iota16():
    return lax.iota(jnp.int32, 16)


def _lane(v, i):
    return jnp.max(jnp.where(_iota16() == i, v, 0))


def _part_body(src_h, dst_h, psrc_h, pdst_h, nch_h,
               stage_s, stage_d, bq_s, bq_d, cnt_v):
    c = lax.axis_index("c")
    s = lax.axis_index("s")
    w = s * NC + c
    base = w * EW
    io = _iota16()
    for r in range(NR):
        rs = RS[r]
        pltpu.sync_copy(src_h.at[pl.ds(base + RO[r], rs)], stage_s.at[pl.ds(0, rs)])
        pltpu.sync_copy(dst_h.at[pl.ds(base + RO[r], rs)], stage_d.at[pl.ds(0, rs)])
        nv = (rs + 15) // 16

        def vbody(i, ptrs, rs=rs):
            sv = stage_s[pl.ds(i * 16, 16)]
            dv = stage_d[pl.ds(i * 16, 16)]
            valid = (i * 16 + io) < rs
            one = jnp.full((16,), 1, jnp.int32)
            zero = jnp.full((16,), 0, jnp.int32)
            qv = (jnp.where(dv >= QSTEP, one, zero)
                  + jnp.where(dv >= 2 * QSTEP, one, zero)
                  + jnp.where(dv >= 3 * QSTEP, one, zero))
            localv = dv - QSTEP * qv
            qq = jnp.where(valid, qv, jnp.full((16,), -1, jnp.int32))
            out = []
            for q in range(4):
                m = qq == q
                p = ptrs[q]
                mi = jnp.where(m, one, zero)
                cum = plsc.cumsum(mi)
                pos = jnp.where(m, q * CAPR + p + (cum - mi), 4 * CAPR + io)
                plsc.store_scatter(bq_s, [pos], sv)
                plsc.store_scatter(bq_d, [pos], localv)
                out.append(p + jnp.sum(mi))
            return tuple(out)

        ptrs = lax.fori_loop(0, nv, vbody, (0, 0, 0, 0))
        cv = jnp.zeros((16,), jnp.int32)
        for q in range(4):
            p = ptrs[q]
            for j in range(8):
                lanes = (io + 16 * j) & 63
                bq_s[pl.ds(q * CAPR + p + 16 * j, 16)] = lanes
                bq_d[pl.ds(q * CAPR + p + 16 * j, 16)] = DUMMY + lanes
            cv = cv + jnp.where(io == q, (p + CHUNK - 1) // CHUNK, 0)
        cnt_v[...] = cv
        pltpu.sync_copy(cnt_v, nch_h.at[w, r])
        pltpu.sync_copy(bq_s.at[pl.ds(0, 4 * CAPR)], psrc_h.at[w, r])
        pltpu.sync_copy(bq_d.at[pl.ds(0, 4 * CAPR)], pdst_h.at[w, r])


_k_part = functools.partial(
    pl.kernel, _part_body,
    out_type=(jax.ShapeDtypeStruct((NW, NR, 4 * CAPR), jnp.int32),
              jax.ShapeDtypeStruct((NW, NR, 4 * CAPR), jnp.int32),
              jax.ShapeDtypeStruct((NW, NR, 16), jnp.int32)),
    mesh=_mesh, compiler_params=_sc_params,
    scratch_types=[pltpu.VMEM((RS[0],), jnp.int32),
                   pltpu.VMEM((RS[0],), jnp.int32),
                   pltpu.VMEM((4 * CAPR + 16,), jnp.int32),
                   pltpu.VMEM((4 * CAPR + 16,), jnp.int32),
                   pltpu.VMEM((16,), jnp.int32)])


def _deg_body(dstp_h, deg_h, acc_sh, stage_d, idxb, ones_v, zbuf, sem):
    c = lax.axis_index("c")
    s = lax.axis_index("s")
    io = _iota16()
    for j in range(8):
        ones_v[pl.ds(16 * j, 16)] = jnp.full((16,), 1.0, jnp.float32)

    def zfill(i, _):
        zbuf[pl.ds(i * 16, 16)] = jnp.zeros((16,), jnp.float32)
        return 0

    lax.fori_loop(0, 100, zfill, 0)
    zs = DEG_ACC // NS
    pltpu.sync_copy(zbuf.at[pl.ds(0, zs)], acc_sh.at[pl.ds(s * zs, zs)])
    plsc.subcore_barrier()
    half_sz = jnp.where(c == 0, HALF, N - HALF)
    cbase = c * HALF

    def blk(b, _):
        pltpu.sync_copy(dstp_h.at[pl.ds(s * DEG_EW + b * (DEG_BLK * CHUNK),
                                        DEG_BLK * CHUNK)], stage_d)

        def chunk(ch, _):
            for v in range(8):
                d = stage_d[pl.ds(ch * CHUNK + 16 * v, 16)]
                local = d - cbase
                ok = (local >= 0) & (local < half_sz)
                idxb[ch, pl.ds(16 * v, 16)] = jnp.where(
                    ok, local, HALF + ((io + 16 * (v + ch)) & 63))
            return 0

        lax.fori_loop(0, DEG_BLK, chunk, 0)
        descs = [pltpu.async_copy(ones_v, acc_sh.at[idxb.at[ch]], sem, add=True)
                 for ch in range(DEG_BLK)]
        for dsc in descs:
            dsc.wait()
        return 0

    lax.fori_loop(0, DEG_NB, blk, 0)
    plsc.subcore_barrier()
    wsz = HALF // NS
    pltpu.sync_copy(acc_sh.at[pl.ds(s * wsz, wsz)], zbuf.at[pl.ds(0, wsz)])
    pltpu.sync_copy(zbuf.at[pl.ds(0, wsz)],
                    deg_h.at[pl.ds(c * HALF + s * wsz, wsz)])


_k_deg = functools.partial(
    pl.kernel, _deg_body,
    out_type=jax.ShapeDtypeStruct((2 * HALF,), jnp.float32),
    mesh=_mesh, compiler_params=_sc_params,
    scratch_types=[pltpu.VMEM_SHARED((DEG_ACC,), jnp.float32),
                   pltpu.VMEM((DEG_BLK * CHUNK,), jnp.int32),
                   pltpu.VMEM((DEG_BLK, CHUNK), jnp.int32),
                   pltpu.VMEM((CHUNK,), jnp.float32),
                   pltpu.VMEM((1600,), jnp.float32),
                   pltpu.SemaphoreType.DMA])


def _agg_body(table_h, psrc_h, pdst_h, nch_h, outz_h,
              acc_sh, sidx, didx, rows, nch_v, zbuf, gsem, isem, F):
    c = lax.axis_index("c")
    s = lax.axis_index("s")
    io = _iota16()

    def zfill(i, _):
        for v in range(F // 16):
            zbuf[i, pl.ds(16 * v, 16)] = jnp.zeros((16,), jnp.float32)
        return 0

    lax.fori_loop(0, 56, zfill, 0)
    for p in range(2):
        q = 2 * c + p
        for t in range(14):
            pltpu.sync_copy(zbuf.at[pl.ds(0, 56)],
                            acc_sh.at[pl.ds(s * 792 + t * 56, 56)])
        pltpu.sync_copy(zbuf.at[pl.ds(0, 8)],
                        acc_sh.at[pl.ds(s * 792 + 784, 8)])
        plsc.subcore_barrier()
        for pwi in range(2):
            pw = 2 * s + pwi
            for r in range(NR):
                pltpu.sync_copy(nch_h.at[pw, r], nch_v)
                nc = _lane(nch_v[...], q)

                nc2 = nc * 2

                @pl.when(nc2 > 0)
                def _(pw=pw, r=r, q=q, nc2=nc2):
                    HC = CHUNK // 2
                    pltpu.async_copy(
                        psrc_h.at[pw, r, pl.ds(q * CAPR, HC)],
                        sidx.at[0], isem)
                    pltpu.async_copy(
                        pdst_h.at[pw, r, pl.ds(q * CAPR, HC)],
                        didx.at[0], isem)

                    def chunk(j, _):
                        par = j & 1
                        pltpu.make_async_copy(
                            psrc_h.at[pw, r, pl.ds(q * CAPR, HC)],
                            sidx.at[par], isem).wait()
                        pltpu.make_async_copy(
                            pdst_h.at[pw, r, pl.ds(q * CAPR, HC)],
                            didx.at[par], isem).wait()
                        gd = pltpu.async_copy(table_h.at[sidx.at[par]],
                                              rows.at[par], gsem)

                        @pl.when(j > 0)
                        def _():
                            pltpu.sync_copy(rows.at[1 - par],
                                            acc_sh.at[didx.at[1 - par]],
                                            add=True)

                        @pl.when(j + 1 < nc2)
                        def _():
                            nxt = 1 - par
                            off = q * CAPR + (j + 1) * HC
                            pltpu.async_copy(
                                psrc_h.at[pw, r, pl.ds(off, HC)],
                                sidx.at[nxt], isem)
                            pltpu.async_copy(
                                pdst_h.at[pw, r, pl.ds(off, HC)],
                                didx.at[nxt], isem)

                        gd.wait()
                        return 0

                    lax.fori_loop(0, nc2, chunk, 0)
                    last = (nc2 - 1) & 1
                    pltpu.sync_copy(rows.at[last], acc_sh.at[didx.at[last]],
                                    add=True)
        plsc.subcore_barrier()
        qbase = q * QSTEP

        @pl.when(q < 3)
        def _():
            for t in range(14):
                pltpu.sync_copy(acc_sh.at[pl.ds(s * WB + t * 56, 56)],
                                zbuf.at[pl.ds(0, 56)])
                pltpu.sync_copy(zbuf.at[pl.ds(0, 56)],
                                outz_h.at[pl.ds(qbase + s * WB + t * 56, 56)])

        @pl.when(q == 3)
        def _():
            for t in range(13):
                pltpu.sync_copy(acc_sh.at[pl.ds(s * WB3 + t * 56, 56)],
                                zbuf.at[pl.ds(0, 56)])
                pltpu.sync_copy(zbuf.at[pl.ds(0, 56)],
                                outz_h.at[pl.ds(qbase + s * WB3 + t * 56, 56)])
            pltpu.sync_copy(acc_sh.at[pl.ds(s * WB3 + 728, 48)],
                            zbuf.at[pl.ds(0, 48)])
            pltpu.sync_copy(zbuf.at[pl.ds(0, 48)],
                            outz_h.at[pl.ds(qbase + s * WB3 + 728, 48)])

        plsc.subcore_barrier()
        lax.fori_loop(0, 56, zfill, 0)


def _make_agg(F):
    return functools.partial(
        pl.kernel, functools.partial(_agg_body, F=F),
        out_type=jax.ShapeDtypeStruct((NPAD, F), jnp.float32),
        mesh=_mesh, compiler_params=_sc_params,
        scratch_types=[pltpu.VMEM_SHARED((ACC_R, F), jnp.float32),
                       pltpu.VMEM((2, CHUNK // 2), jnp.int32),
                       pltpu.VMEM((2, CHUNK // 2), jnp.int32),
                       pltpu.VMEM((2, CHUNK // 2, F), jnp.float32),
                       pltpu.VMEM((16,), jnp.int32),
                       pltpu.VMEM((56, F), jnp.float32),
                       pltpu.SemaphoreType.DMA,
                       pltpu.SemaphoreType.DMA])


_k_agg128 = _make_agg(128)


def _segmax_body(xg1, xg2, h1, h2, h3, off_h, out_h, stg, offs_v, outrows, sem):
    c = lax.axis_index("c")
    s = lax.axis_index("s")
    w = s * NC + c
    io = _iota16()
    pltpu.sync_copy(off_h.at[pl.ds(16 * w, 32)], offs_v)
    v0 = offs_v[pl.ds(0, 16)]
    v1 = offs_v[pl.ds(16, 16)]
    arrs = (xg1, xg2, h1, h2, h3)

    def seg(gi, _):
        a = jnp.max(jnp.where(io == gi, v0, 0))
        b = (jnp.max(jnp.where(io == gi + 1, v0, 0))
             + jnp.max(jnp.where(io == gi - 15, v1, 0)))
        a8 = a - (a & 7)
        nb = (b - a8 + 63) // 64

        acc0 = tuple(jnp.full((16,), FMIN, jnp.float32) for _ in range(36))

        def blk(k, acc):
            start = pl.multiple_of(a8 + 64 * k, 8)
            descs = [pltpu.async_copy(arrs[t].at[pl.ds(start, 64)],
                                      stg.at[t], sem) for t in range(5)]
            for dsc in descs:
                dsc.wait()
            lo = jnp.maximum(a - start, 0)
            lim = jnp.minimum(64, b - start)

            def row(i, acc):
                new = []
                for v in range(4):
                    g1 = stg[0, i, pl.ds(16 * v, 16)]
                    g2 = stg[1, i, pl.ds(16 * v, 16)]
                    a1 = stg[2, i, pl.ds(16 * v, 16)]
                    a2 = stg[3, i, pl.ds(16 * v, 16)]
                    a3 = stg[4, i, pl.ds(16 * v, 16)]
                    combos = (g1, g2, g1 + g2, g1 * g2, a1, a2, a3,
                              a1 + a2 + a3, a1 * a2 * a3)
                    for k9 in range(9):
                        new.append(jnp.maximum(acc[k9 * 4 + v], combos[k9]))
                out = [None] * 36
                idx = 0
                for v in range(4):
                    for k9 in range(9):
                        out[k9 * 4 + v] = new[idx]
                        idx += 1
                return tuple(out)

            return lax.fori_loop(lo, lim, row, acc)

        acc = lax.fori_loop(0, nb, blk, acc0)
        for k36 in range(36):
            outrows[gi, pl.ds(16 * k36, 16)] = acc[k36]
        return 0

    lax.fori_loop(0, 16, seg, 0)
    pltpu.sync_copy(outrows, out_h.at[pl.ds(pl.multiple_of(16 * w, 8), 16)])


_k_segmax = functools.partial(
    pl.kernel, _segmax_body,
    out_type=jax.ShapeDtypeStruct((G, 9 * D), jnp.float32),
    mesh=_mesh, compiler_params=_sc_params,
    scratch_types=[pltpu.VMEM((5, 64, D), jnp.float32),
                   pltpu.VMEM((32,), jnp.int32),
                   pltpu.VMEM((16, 9 * D), jnp.float32),
                   pltpu.SemaphoreType.DMA])


TB = 2000
NTB = N // TB
def _dot(a, b):
    return jax.lax.dot_general(a, b, (((1,), (0,)), ((), ())),
                               preferred_element_type=jnp.float32)


def _tc1_body(x_ref, deg_ref, w1_ref, dis_ref, t1_ref):
    di = 1.0 / jnp.sqrt(deg_ref[...] + 1.0)
    dis_ref[...] = di
    t1_ref[:, 0:64] = _dot(x_ref[...], w1_ref[...]) * di
    t1_ref[:, 64:128] = x_ref[:, 0:64]


def _tc2_body(z1_ref, t1_ref, dis_ref, x_ref, bg1_ref, wg2_ref,
              xg1_ref, t2_ref):
    di = dis_ref[...]
    xg1 = jax.nn.relu(di * (z1_ref[:, 0:64] + t1_ref[:, 0:64]) + bg1_ref[...])
    xg1_ref[...] = xg1
    t2_ref[:, 0:64] = di * _dot(xg1, wg2_ref[...])
    t2_ref[:, 64:128] = x_ref[:, 64:128]


def _tc3_body(z1_ref, z2_ref, t2_ref, dis_ref, x_ref, bg2_ref,
              g0w1_ref, g0b1_ref, g0w2_ref, g0b2_ref,
              xg2_ref, q0_ref, sums_ref):
    i = pl.program_id(0)
    agg0 = jnp.concatenate([z1_ref[:, 64:128] + x_ref[:, 0:64],
                            z2_ref[:, 64:128] + x_ref[:, 64:128]], axis=1)
    mid0 = jax.nn.relu(_dot(agg0, g0w1_ref[...]) + g0b1_ref[...])
    q0 = jax.nn.relu(_dot(mid0, g0w2_ref[...]) + g0b2_ref[...])
    q0_ref[...] = q0
    xg2_ref[...] = jax.nn.relu(
        dis_ref[...] * (z2_ref[:, 0:64] + t2_ref[:, 0:64]) + bg2_ref[...])

    @pl.when(i == 0)
    def _():
        sums_ref[...] = jnp.zeros_like(sums_ref)

    sums_ref[...] += jnp.stack([jnp.sum(q0, axis=0), jnp.sum(q0 * q0, axis=0)])


def _bn_cols(q, sums, g, b):
    m = sums[0:1, :] / N
    var = sums[1:2, :] / N - m * m
    return g * (q - m) / jnp.sqrt(var + 1e-5) + b


def _tcbn_body(q_ref, sums_ref, bng_ref, bnb_ref, h_ref, t_ref):
    h = _bn_cols(q_ref[...], sums_ref[...], bng_ref[...], bnb_ref[...])
    h_ref[...] = h
    t_ref[:, 0:64] = h
    t_ref[:, 64:128] = jnp.zeros((q_ref.shape[0], 64), jnp.float32)


def _tcgin_body(z_ref, h_ref, gb1_ref, gw1_ref, gw2_ref, gb2_ref,
                q_ref, sums_ref):
    i = pl.program_id(0)
    agg = z_ref[:, 0:64] + h_ref[...]
    mid = jax.nn.relu(_dot(agg, gw1_ref[...]) + gb1_ref[...])
    q = jax.nn.relu(_dot(mid, gw2_ref[...]) + gb2_ref[...])
    q_ref[...] = q

    @pl.when(i == 0)
    def _():
        sums_ref[...] = jnp.zeros_like(sums_ref)

    sums_ref[...] += jnp.stack([jnp.sum(q, axis=0), jnp.sum(q * q, axis=0)])


def _tcbn3_body(q_ref, sums_ref, bng_ref, bnb_ref, h_ref):
    h_ref[...] = _bn_cols(q_ref[...], sums_ref[...], bng_ref[...], bnb_ref[...])


def _tcoff_body(b_ref, off_ref):
    i = pl.program_id(0)
    thr = lax.broadcasted_iota(jnp.int32, (528, 1), 0)
    cmp = (b_ref[0] < thr).astype(jnp.int32)
    ps = jnp.sum(cmp, axis=1)[None, :]

    @pl.when(i == 0)
    def _():
        off_ref[...] = jnp.zeros_like(off_ref)

    off_ref[...] += jnp.broadcast_to(ps, off_ref.shape)


def _row_spec(width):
    return pl.BlockSpec((TB, width), lambda i: (i, 0))


def _full_spec(shape):
    return pl.BlockSpec(shape, lambda i: tuple(0 for _ in shape))


def _mk_tc(body, in_widths, out_widths, consts, out_consts):
    in_specs = [_row_spec(w) for w in in_widths] + [_full_spec(s) for s in consts]
    out_specs = ([_row_spec(w) for w in out_widths]
                 + [_full_spec(s) for s in out_consts])
    out_shape = ([jax.ShapeDtypeStruct((N, w), jnp.float32) for w in out_widths]
                 + [jax.ShapeDtypeStruct(s, jnp.float32) for s in out_consts])
    return pl.pallas_call(body, grid=(NTB,), in_specs=in_specs,
                          out_specs=out_specs, out_shape=out_shape)


def kernel(x, edge_index, batch, W_gcn1, b_gcn1, W_gcn2, b_gcn2,
           gin0_W1, gin0_b1, gin0_W2, gin0_b2,
           gin1_W1, gin1_b1, gin1_W2, gin1_b2,
           gin2_W1, gin2_b1, gin2_W2, gin2_b2,
           bn0_g, bn0_b, bn1_g, bn1_b, bn2_g, bn2_b):
    f32 = jnp.float32
    src = edge_index[0]
    dst = edge_index[1]

    x_pad = jnp.pad(x, ((0, 0), (0, 128 - x.shape[1])))
    w1p = jnp.pad(W_gcn1, ((0, 128 - W_gcn1.shape[0]), (0, 0)))
    w0p = jnp.pad(gin0_W1, ((0, 128 - gin0_W1.shape[0]), (0, 0)))
    dst_pad = jnp.concatenate(
        [dst, jnp.full((16 * DEG_EW - E,), jnp.int32(2 ** 30))])
    batch2d = batch.reshape(50, 1000)
    b_gcn1r = b_gcn1.reshape(1, 64)
    b_gcn2r = b_gcn2.reshape(1, 64)
    g0b1 = gin0_b1.reshape(1, 64)
    g0b2 = gin0_b2.reshape(1, 64)
    g1b1 = gin1_b1.reshape(1, 64)
    g1b2 = gin1_b2.reshape(1, 64)
    g2b1 = gin2_b1.reshape(1, 64)
    g2b2 = gin2_b2.reshape(1, 64)

    psrc, pdst, nch = _k_part()(src, dst)
    _DEBUG = 0
    if _DEBUG == 1:
        return jnp.zeros((G, 9 * D), f32) + nch.sum().astype(f32)
    deg = _k_deg()(dst_pad)[:N]
    if _DEBUG == 2:
        return jnp.zeros((G, 9 * D), f32) + deg.sum()

    dis, t1 = _mk_tc(_tc1_body, [128, 1], [1, 128], [(128, 64)], [])(
        x_pad, deg.reshape(N, 1), w1p)

    z1 = _k_agg128()(t1, psrc, pdst, nch)[:N]

    xg1, t2 = _mk_tc(_tc2_body, [128, 128, 1, 128], [64, 128],
                     [(1, 64), (64, 64)], [])(
        z1, t1, dis, x_pad, b_gcn1r, W_gcn2)

    z2 = _k_agg128()(t2, psrc, pdst, nch)[:N]

    xg2, q0, sums0 = _mk_tc(
        _tc3_body, [128, 128, 128, 1, 128], [64, 64],
        [(1, 64), (128, 64), (1, 64), (64, 64), (1, 64)], [(2, 64)])(
        z1, z2, t2, dis, x_pad, b_gcn2r, w0p, g0b1, gin0_W2, g0b2)

    h1, t3 = _mk_tc(_tcbn_body, [64], [64, 128], [(2, 64), (1, 64), (1, 64)],
                    [])(q0, sums0, bn0_g.reshape(1, 64), bn0_b.reshape(1, 64))

    z3 = _k_agg128()(t3, psrc, pdst, nch)[:N]

    q1, sums1 = _mk_tc(_tcgin_body, [128, 64], [64],
                       [(1, 64), (64, 64), (64, 64), (1, 64)], [(2, 64)])(
        z3, h1, g1b1, gin1_W1, gin1_W2, g1b2)

    h2, t4 = _mk_tc(_tcbn_body, [64], [64, 128], [(2, 64), (1, 64), (1, 64)],
                    [])(q1, sums1, bn1_g.reshape(1, 64), bn1_b.reshape(1, 64))

    z4 = _k_agg128()(t4, psrc, pdst, nch)[:N]

    q2, sums2 = _mk_tc(_tcgin_body, [128, 64], [64],
                       [(1, 64), (64, 64), (64, 64), (1, 64)], [(2, 64)])(
        z4, h2, g2b1, gin2_W1, gin2_W2, g2b2)

    (h3,) = _mk_tc(_tcbn3_body, [64], [64], [(2, 64), (1, 64), (1, 64)], [])(
        q2, sums2, bn2_g.reshape(1, 64), bn2_b.reshape(1, 64))

    off2d = pl.pallas_call(
        _tcoff_body, grid=(50,),
        in_specs=[pl.BlockSpec((1, 1, 1000), lambda i: (i, 0, 0))],
        out_specs=pl.BlockSpec((8, 528), lambda i: (0, 0)),
        out_shape=jax.ShapeDtypeStruct((8, 528), jnp.int32))(
        batch2d.reshape(50, 1, 1000))
    off = off2d[0]

    pad = ((0, 112), (0, 0))
    out = _k_segmax()(jnp.pad(xg1, pad), jnp.pad(xg2, pad), jnp.pad(h1, pad),
                      jnp.pad(h2, pad), jnp.pad(h3, pad), off)
    return out


# speedup vs baseline: 1.0133x; 1.0133x over previous
import functools

import jax
import jax.numpy as jnp
from jax import lax
from jax.experimental import pallas as pl
from jax.experimental.pallas import tpu as pltpu
from jax.experimental.pallas import tpu_sc as plsc

N = 50000
E = 800000
D = 64
G = 512
NC = 2
NS = 16
NW = NC * NS
L = 16

QSTEP = 12544
QS_LAST = N - 3 * QSTEP
DUMMY = QSTEP
ACC_R = 12672
WB = QSTEP // NS
WB3 = 776
NPAD = 3 * QSTEP + NS * WB3

EW = E // NW
RS = (6272, 6272, 6272, 6184)
RO = (0, 6272, 12544, 18816)
CAPR = 6400
NR = 4
CHUNK = 128

DEG_EW = 50048
DEG_BLK = 23
DEG_NB = 17
HALF = 2 * QSTEP
DEG_ACC = HALF + 128

FMIN = -3.0e38

_mesh = plsc.VectorSubcoreMesh(core_axis_name="c", subcore_axis_name="s",
                               num_cores=NC, num_subcores=NS)
_sc_params = pltpu.CompilerParams(needs_layout_passes=False)


def _iota16():
    return lax.iota(jnp.int32, 16)


def _lane(v, i):
    return jnp.max(jnp.where(_iota16() == i, v, 0))


def _part_body(src_h, dst_h, psrc_h, pdst_h, nch_h,
               stage_s, stage_d, bq_s, bq_d, cnt_v):
    c = lax.axis_index("c")
    s = lax.axis_index("s")
    w = s * NC + c
    base = w * EW
    io = _iota16()
    for r in range(NR):
        rs = RS[r]
        pltpu.sync_copy(src_h.at[pl.ds(base + RO[r], rs)], stage_s.at[pl.ds(0, rs)])
        pltpu.sync_copy(dst_h.at[pl.ds(base + RO[r], rs)], stage_d.at[pl.ds(0, rs)])
        nv = (rs + 15) // 16

        def vbody(i, ptrs, rs=rs):
            sv = stage_s[pl.ds(i * 16, 16)]
            dv = stage_d[pl.ds(i * 16, 16)]
            valid = (i * 16 + io) < rs
            one = jnp.full((16,), 1, jnp.int32)
            zero = jnp.full((16,), 0, jnp.int32)
            qv = (jnp.where(dv >= QSTEP, one, zero)
                  + jnp.where(dv >= 2 * QSTEP, one, zero)
                  + jnp.where(dv >= 3 * QSTEP, one, zero))
            localv = dv - QSTEP * qv
            qq = jnp.where(valid, qv, jnp.full((16,), -1, jnp.int32))
            out = []
            for q in range(4):
                m = qq == q
                p = ptrs[q]
                mi = jnp.where(m, one, zero)
                cum = plsc.cumsum(mi)
                pos = jnp.where(m, q * CAPR + p + (cum - mi), 4 * CAPR + io)
                plsc.store_scatter(bq_s, [pos], sv)
                plsc.store_scatter(bq_d, [pos], localv)
                out.append(p + jnp.sum(mi))
            return tuple(out)

        ptrs = lax.fori_loop(0, nv, vbody, (0, 0, 0, 0))
        cv = jnp.zeros((16,), jnp.int32)
        for q in range(4):
            p = ptrs[q]
            for j in range(8):
                lanes = (io + 16 * j) & 63
                bq_s[pl.ds(q * CAPR + p + 16 * j, 16)] = lanes
                bq_d[pl.ds(q * CAPR + p + 16 * j, 16)] = DUMMY + lanes
            cv = cv + jnp.where(io == q, (p + CHUNK - 1) // CHUNK, 0)
        cnt_v[...] = cv
        pltpu.sync_copy(cnt_v, nch_h.at[w, r])
        pltpu.sync_copy(bq_s.at[pl.ds(0, 4 * CAPR)], psrc_h.at[w, r])
        pltpu.sync_copy(bq_d.at[pl.ds(0, 4 * CAPR)], pdst_h.at[w, r])


_k_part = functools.partial(
    pl.kernel, _part_body,
    out_type=(jax.ShapeDtypeStruct((NW, NR, 4 * CAPR), jnp.int32),
              jax.ShapeDtypeStruct((NW, NR, 4 * CAPR), jnp.int32),
              jax.ShapeDtypeStruct((NW, NR, 16), jnp.int32)),
    mesh=_mesh, compiler_params=_sc_params,
    scratch_types=[pltpu.VMEM((RS[0],), jnp.int32),
                   pltpu.VMEM((RS[0],), jnp.int32),
                   pltpu.VMEM((4 * CAPR + 16,), jnp.int32),
                   pltpu.VMEM((4 * CAPR + 16,), jnp.int32),
                   pltpu.VMEM((16,), jnp.int32)])


def _deg_body(dstp_h, deg_h, acc_sh, stage_d, idxb, ones_v, zbuf, sem):
    c = lax.axis_index("c")
    s = lax.axis_index("s")
    io = _iota16()
    for j in range(8):
        ones_v[pl.ds(16 * j, 16)] = jnp.full((16,), 1.0, jnp.float32)

    def zfill(i, _):
        zbuf[pl.ds(i * 16, 16)] = jnp.zeros((16,), jnp.float32)
        return 0

    lax.fori_loop(0, 100, zfill, 0)
    zs = DEG_ACC // NS
    pltpu.sync_copy(zbuf.at[pl.ds(0, zs)], acc_sh.at[pl.ds(s * zs, zs)])
    plsc.subcore_barrier()
    half_sz = jnp.where(c == 0, HALF, N - HALF)
    cbase = c * HALF

    def blk(b, _):
        pltpu.sync_copy(dstp_h.at[pl.ds(s * DEG_EW + b * (DEG_BLK * CHUNK),
                                        DEG_BLK * CHUNK)], stage_d)

        def chunk(ch, _):
            for v in range(8):
                d = stage_d[pl.ds(ch * CHUNK + 16 * v, 16)]
                local = d - cbase
                ok = (local >= 0) & (local < half_sz)
                idxb[ch, pl.ds(16 * v, 16)] = jnp.where(
                    ok, local, HALF + ((io + 16 * (v + ch)) & 63))
            return 0

        lax.fori_loop(0, DEG_BLK, chunk, 0)
        descs = [pltpu.async_copy(ones_v, acc_sh.at[idxb.at[ch]], sem, add=True)
                 for ch in range(DEG_BLK)]
        for dsc in descs:
            dsc.wait()
        return 0

    lax.fori_loop(0, DEG_NB, blk, 0)
    plsc.subcore_barrier()
    wsz = HALF // NS
    pltpu.sync_copy(acc_sh.at[pl.ds(s * wsz, wsz)], zbuf.at[pl.ds(0, wsz)])
    pltpu.sync_copy(zbuf.at[pl.ds(0, wsz)],
                    deg_h.at[pl.ds(c * HALF + s * wsz, wsz)])


_k_deg = functools.partial(
    pl.kernel, _deg_body,
    out_type=jax.ShapeDtypeStruct((2 * HALF,), jnp.float32),
    mesh=_mesh, compiler_params=_sc_params,
    scratch_types=[pltpu.VMEM_SHARED((DEG_ACC,), jnp.float32),
                   pltpu.VMEM((DEG_BLK * CHUNK,), jnp.int32),
                   pltpu.VMEM((DEG_BLK, CHUNK), jnp.int32),
                   pltpu.VMEM((CHUNK,), jnp.float32),
                   pltpu.VMEM((1600,), jnp.float32),
                   pltpu.SemaphoreType.DMA])


def _agg_body(table_h, psrc_h, pdst_h, nch_h, outz_h,
              acc_sh, sidx, didx, rows, nch_v, zbuf, gsem, isem, F):
    c = lax.axis_index("c")
    s = lax.axis_index("s")
    io = _iota16()

    def zfill(i, _):
        for v in range(F // 16):
            zbuf[i, pl.ds(16 * v, 16)] = jnp.zeros((16,), jnp.float32)
        return 0

    lax.fori_loop(0, 56, zfill, 0)
    for p in range(2):
        q = 2 * c + p
        for t in range(14):
            pltpu.sync_copy(zbuf.at[pl.ds(0, 56)],
                            acc_sh.at[pl.ds(s * 792 + t * 56, 56)])
        pltpu.sync_copy(zbuf.at[pl.ds(0, 8)],
                        acc_sh.at[pl.ds(s * 792 + 784, 8)])
        plsc.subcore_barrier()
        for pwi in range(2):
            pw = 2 * s + pwi
            for r in range(NR):
                pltpu.sync_copy(nch_h.at[pw, r], nch_v)
                nc = _lane(nch_v[...], q)

                @pl.when(nc > 0)
                def _(pw=pw, r=r, q=q, nc=nc):
                    pltpu.async_copy(
                        psrc_h.at[pw, r, pl.ds(q * CAPR, CHUNK)],
                        sidx.at[0], isem)
                    pltpu.async_copy(
                        pdst_h.at[pw, r, pl.ds(q * CAPR, CHUNK)],
                        didx.at[0], isem)

                    def chunk(j, _):
                        par = j & 1
                        pltpu.make_async_copy(
                            psrc_h.at[pw, r, pl.ds(q * CAPR, CHUNK)],
                            sidx.at[par], isem).wait()
                        pltpu.make_async_copy(
                            pdst_h.at[pw, r, pl.ds(q * CAPR, CHUNK)],
                            didx.at[par], isem).wait()
                        gd = pltpu.async_copy(table_h.at[sidx.at[par]],
                                              rows, gsem)

                        @pl.when(j + 1 < nc)
                        def _():
                            nxt = 1 - par
                            off = q * CAPR + (j + 1) * CHUNK
                            pltpu.async_copy(
                                psrc_h.at[pw, r, pl.ds(off, CHUNK)],
                                sidx.at[nxt], isem)
                            pltpu.async_copy(
                                pdst_h.at[pw, r, pl.ds(off, CHUNK)],
                                didx.at[nxt], isem)

                        gd.wait()
                        pltpu.sync_copy(rows, acc_sh.at[didx.at[par]],
                                        add=True)
                        return 0

                    lax.fori_loop(0, nc, chunk, 0)
        plsc.subcore_barrier()
        qbase = q * QSTEP

        @pl.when(q < 3)
        def _():
            for t in range(14):
                pltpu.sync_copy(acc_sh.at[pl.ds(s * WB + t * 56, 56)],
                                zbuf.at[pl.ds(0, 56)])
                pltpu.sync_copy(zbuf.at[pl.ds(0, 56)],
                                outz_h.at[pl.ds(qbase + s * WB + t * 56, 56)])

        @pl.when(q == 3)
        def _():
            for t in range(13):
                pltpu.sync_copy(acc_sh.at[pl.ds(s * WB3 + t * 56, 56)],
                                zbuf.at[pl.ds(0, 56)])
                pltpu.sync_copy(zbuf.at[pl.ds(0, 56)],
                                outz_h.at[pl.ds(qbase + s * WB3 + t * 56, 56)])
            pltpu.sync_copy(acc_sh.at[pl.ds(s * WB3 + 728, 48)],
                            zbuf.at[pl.ds(0, 48)])
            pltpu.sync_copy(zbuf.at[pl.ds(0, 48)],
                            outz_h.at[pl.ds(qbase + s * WB3 + 728, 48)])

        plsc.subcore_barrier()
        lax.fori_loop(0, 56, zfill, 0)


def _make_agg(F):
    return functools.partial(
        pl.kernel, functools.partial(_agg_body, F=F),
        out_type=jax.ShapeDtypeStruct((NPAD, F), jnp.float32),
        mesh=_mesh, compiler_params=_sc_params,
        scratch_types=[pltpu.VMEM_SHARED((ACC_R, F), jnp.float32),
                       pltpu.VMEM((2, CHUNK), jnp.int32),
                       pltpu.VMEM((2, CHUNK), jnp.int32),
                       pltpu.VMEM((CHUNK, F), jnp.float32),
                       pltpu.VMEM((16,), jnp.int32),
                       pltpu.VMEM((56, F), jnp.float32),
                       pltpu.SemaphoreType.DMA,
                       pltpu.SemaphoreType.DMA])


_k_agg128 = _make_agg(128)


def _segmax_body(xg1, xg2, h1, h2, h3, off_h, out_h, stg, offs_v, outrows, sem):
    c = lax.axis_index("c")
    s = lax.axis_index("s")
    w = s * NC + c
    io = _iota16()
    pltpu.sync_copy(off_h.at[pl.ds(16 * w, 32)], offs_v)
    v0 = offs_v[pl.ds(0, 16)]
    v1 = offs_v[pl.ds(16, 16)]
    arrs = (xg1, xg2, h1, h2, h3)

    def seg(gi, _):
        a = jnp.max(jnp.where(io == gi, v0, 0))
        b = (jnp.max(jnp.where(io == gi + 1, v0, 0))
             + jnp.max(jnp.where(io == gi - 15, v1, 0)))
        a8 = a - (a & 7)
        nb = (b - a8 + 63) // 64

        acc0 = tuple(jnp.full((16,), FMIN, jnp.float32) for _ in range(36))

        def blk(k, acc):
            start = pl.multiple_of(a8 + 64 * k, 8)
            descs = [pltpu.async_copy(arrs[t].at[pl.ds(start, 64)],
                                      stg.at[t], sem) for t in range(5)]
            for dsc in descs:
                dsc.wait()
            lo = jnp.maximum(a - start, 0)
            lim = jnp.minimum(64, b - start)

            def row(i, acc):
                new = []
                for v in range(4):
                    g1 = stg[0, i, pl.ds(16 * v, 16)]
                    g2 = stg[1, i, pl.ds(16 * v, 16)]
                    a1 = stg[2, i, pl.ds(16 * v, 16)]
                    a2 = stg[3, i, pl.ds(16 * v, 16)]
                    a3 = stg[4, i, pl.ds(16 * v, 16)]
                    combos = (g1, g2, g1 + g2, g1 * g2, a1, a2, a3,
                              a1 + a2 + a3, a1 * a2 * a3)
                    for k9 in range(9):
                        new.append(jnp.maximum(acc[k9 * 4 + v], combos[k9]))
                out = [None] * 36
                idx = 0
                for v in range(4):
                    for k9 in range(9):
                        out[k9 * 4 + v] = new[idx]
                        idx += 1
                return tuple(out)

            return lax.fori_loop(lo, lim, row, acc)

        acc = lax.fori_loop(0, nb, blk, acc0)
        for k36 in range(36):
            outrows[gi, pl.ds(16 * k36, 16)] = acc[k36]
        return 0

    lax.fori_loop(0, 16, seg, 0)
    pltpu.sync_copy(outrows, out_h.at[pl.ds(pl.multiple_of(16 * w, 8), 16)])


_k_segmax = functools.partial(
    pl.kernel, _segmax_body,
    out_type=jax.ShapeDtypeStruct((G, 9 * D), jnp.float32),
    mesh=_mesh, compiler_params=_sc_params,
    scratch_types=[pltpu.VMEM((5, 64, D), jnp.float32),
                   pltpu.VMEM((32,), jnp.int32),
                   pltpu.VMEM((16, 9 * D), jnp.float32),
                   pltpu.SemaphoreType.DMA])


TB = 2000
NTB = N // TB
def _dot(a, b):
    return jax.lax.dot_general(a, b, (((1,), (0,)), ((), ())),
                               preferred_element_type=jnp.float32)


def _tc1_body(x_ref, deg_ref, w1_ref, dis_ref, t1_ref):
    di = 1.0 / jnp.sqrt(deg_ref[...] + 1.0)
    dis_ref[...] = di
    t1_ref[:, 0:64] = _dot(x_ref[...], w1_ref[...]) * di
    t1_ref[:, 64:128] = x_ref[:, 0:64]


def _tc2_body(z1_ref, t1_ref, dis_ref, x_ref, bg1_ref, wg2_ref,
              xg1_ref, t2_ref):
    di = dis_ref[...]
    xg1 = jax.nn.relu(di * (z1_ref[:, 0:64] + t1_ref[:, 0:64]) + bg1_ref[...])
    xg1_ref[...] = xg1
    t2_ref[:, 0:64] = di * _dot(xg1, wg2_ref[...])
    t2_ref[:, 64:128] = x_ref[:, 64:128]


def _tc3_body(z1_ref, z2_ref, t2_ref, dis_ref, x_ref, bg2_ref,
              g0w1_ref, g0b1_ref, g0w2_ref, g0b2_ref,
              xg2_ref, q0_ref, sums_ref):
    i = pl.program_id(0)
    agg0 = jnp.concatenate([z1_ref[:, 64:128] + x_ref[:, 0:64],
                            z2_ref[:, 64:128] + x_ref[:, 64:128]], axis=1)
    mid0 = jax.nn.relu(_dot(agg0, g0w1_ref[...]) + g0b1_ref[...])
    q0 = jax.nn.relu(_dot(mid0, g0w2_ref[...]) + g0b2_ref[...])
    q0_ref[...] = q0
    xg2_ref[...] = jax.nn.relu(
        dis_ref[...] * (z2_ref[:, 0:64] + t2_ref[:, 0:64]) + bg2_ref[...])

    @pl.when(i == 0)
    def _():
        sums_ref[...] = jnp.zeros_like(sums_ref)

    sums_ref[...] += jnp.stack([jnp.sum(q0, axis=0), jnp.sum(q0 * q0, axis=0)])


def _bn_cols(q, sums, g, b):
    m = sums[0:1, :] / N
    var = sums[1:2, :] / N - m * m
    return g * (q - m) / jnp.sqrt(var + 1e-5) + b


def _tcbn_body(q_ref, sums_ref, bng_ref, bnb_ref, h_ref, t_ref):
    h = _bn_cols(q_ref[...], sums_ref[...], bng_ref[...], bnb_ref[...])
    h_ref[...] = h
    t_ref[:, 0:64] = h
    t_ref[:, 64:128] = jnp.zeros((q_ref.shape[0], 64), jnp.float32)


def _tcgin_body(z_ref, h_ref, gb1_ref, gw1_ref, gw2_ref, gb2_ref,
                q_ref, sums_ref):
    i = pl.program_id(0)
    agg = z_ref[:, 0:64] + h_ref[...]
    mid = jax.nn.relu(_dot(agg, gw1_ref[...]) + gb1_ref[...])
    q = jax.nn.relu(_dot(mid, gw2_ref[...]) + gb2_ref[...])
    q_ref[...] = q

    @pl.when(i == 0)
    def _():
        sums_ref[...] = jnp.zeros_like(sums_ref)

    sums_ref[...] += jnp.stack([jnp.sum(q, axis=0), jnp.sum(q * q, axis=0)])


def _tcbn3_body(q_ref, sums_ref, bng_ref, bnb_ref, h_ref):
    h_ref[...] = _bn_cols(q_ref[...], sums_ref[...], bng_ref[...], bnb_ref[...])


def _tcoff_body(b_ref, off_ref):
    i = pl.program_id(0)
    thr = lax.broadcasted_iota(jnp.int32, (528, 1), 0)
    cmp = (b_ref[0] < thr).astype(jnp.int32)
    ps = jnp.sum(cmp, axis=1)[None, :]

    @pl.when(i == 0)
    def _():
        off_ref[...] = jnp.zeros_like(off_ref)

    off_ref[...] += jnp.broadcast_to(ps, off_ref.shape)


def _row_spec(width):
    return pl.BlockSpec((TB, width), lambda i: (i, 0))


def _full_spec(shape):
    return pl.BlockSpec(shape, lambda i: tuple(0 for _ in shape))


def _mk_tc(body, in_widths, out_widths, consts, out_consts):
    in_specs = [_row_spec(w) for w in in_widths] + [_full_spec(s) for s in consts]
    out_specs = ([_row_spec(w) for w in out_widths]
                 + [_full_spec(s) for s in out_consts])
    out_shape = ([jax.ShapeDtypeStruct((N, w), jnp.float32) for w in out_widths]
                 + [jax.ShapeDtypeStruct(s, jnp.float32) for s in out_consts])
    return pl.pallas_call(body, grid=(NTB,), in_specs=in_specs,
                          out_specs=out_specs, out_shape=out_shape)


def kernel(x, edge_index, batch, W_gcn1, b_gcn1, W_gcn2, b_gcn2,
           gin0_W1, gin0_b1, gin0_W2, gin0_b2,
           gin1_W1, gin1_b1, gin1_W2, gin1_b2,
           gin2_W1, gin2_b1, gin2_W2, gin2_b2,
           bn0_g, bn0_b, bn1_g, bn1_b, bn2_g, bn2_b):
    f32 = jnp.float32
    src = edge_index[0]
    dst = edge_index[1]

    x_pad = jnp.pad(x, ((0, 0), (0, 128 - x.shape[1])))
    w1p = jnp.pad(W_gcn1, ((0, 128 - W_gcn1.shape[0]), (0, 0)))
    w0p = jnp.pad(gin0_W1, ((0, 128 - gin0_W1.shape[0]), (0, 0)))
    dst_pad = jnp.concatenate(
        [dst, jnp.full((16 * DEG_EW - E,), jnp.int32(2 ** 30))])
    batch2d = batch.reshape(50, 1000)
    b_gcn1r = b_gcn1.reshape(1, 64)
    b_gcn2r = b_gcn2.reshape(1, 64)
    g0b1 = gin0_b1.reshape(1, 64)
    g0b2 = gin0_b2.reshape(1, 64)
    g1b1 = gin1_b1.reshape(1, 64)
    g1b2 = gin1_b2.reshape(1, 64)
    g2b1 = gin2_b1.reshape(1, 64)
    g2b2 = gin2_b2.reshape(1, 64)

    psrc, pdst, nch = _k_part()(src, dst)
    _DEBUG = 0
    if _DEBUG == 1:
        return jnp.zeros((G, 9 * D), f32) + nch.sum().astype(f32)
    deg = _k_deg()(dst_pad)[:N]
    if _DEBUG == 2:
        return jnp.zeros((G, 9 * D), f32) + deg.sum()

    dis, t1 = _mk_tc(_tc1_body, [128, 1], [1, 128], [(128, 64)], [])(
        x_pad, deg.reshape(N, 1), w1p)

    z1 = _k_agg128()(t1, psrc, pdst, nch)[:N]

    xg1, t2 = _mk_tc(_tc2_body, [128, 128, 1, 128], [64, 128],
                     [(1, 64), (64, 64)], [])(
        z1, t1, dis, x_pad, b_gcn1r, W_gcn2)

    z2 = _k_agg128()(t2, psrc, pdst, nch)[:N]

    xg2, q0, sums0 = _mk_tc(
        _tc3_body, [128, 128, 128, 1, 128], [64, 64],
        [(1, 64), (128, 64), (1, 64), (64, 64), (1, 64)], [(2, 64)])(
        z1, z2, t2, dis, x_pad, b_gcn2r, w0p, g0b1, gin0_W2, g0b2)

    h1, t3 = _mk_tc(_tcbn_body, [64], [64, 128], [(2, 64), (1, 64), (1, 64)],
                    [])(q0, sums0, bn0_g.reshape(1, 64), bn0_b.reshape(1, 64))

    z3 = _k_agg128()(t3, psrc, pdst, nch)[:N]

    q1, sums1 = _mk_tc(_tcgin_body, [128, 64], [64],
                       [(1, 64), (64, 64), (64, 64), (1, 64)], [(2, 64)])(
        z3, h1, g1b1, gin1_W1, gin1_W2, g1b2)

    h2, t4 = _mk_tc(_tcbn_body, [64], [64, 128], [(2, 64), (1, 64), (1, 64)],
                    [])(q1, sums1, bn1_g.reshape(1, 64), bn1_b.reshape(1, 64))

    z4 = _k_agg128()(t4, psrc, pdst, nch)[:N]

    q2, sums2 = _mk_tc(_tcgin_body, [128, 64], [64],
                       [(1, 64), (64, 64), (64, 64), (1, 64)], [(2, 64)])(
        z4, h2, g2b1, gin2_W1, gin2_W2, g2b2)

    (h3,) = _mk_tc(_tcbn3_body, [64], [64], [(2, 64), (1, 64), (1, 64)], [])(
        q2, sums2, bn2_g.reshape(1, 64), bn2_b.reshape(1, 64))

    off2d = pl.pallas_call(
        _tcoff_body, grid=(50,),
        in_specs=[pl.BlockSpec((1, 1, 1000), lambda i: (i, 0, 0))],
        out_specs=pl.BlockSpec((8, 528), lambda i: (0, 0)),
        out_shape=jax.ShapeDtypeStruct((8, 528), jnp.int32))(
        batch2d.reshape(50, 1, 1000))
    off = off2d[0]

    pad = ((0, 112), (0, 0))
    out = _k_segmax()(jnp.pad(xg1, pad), jnp.pad(xg2, pad), jnp.pad(h1, pad),
                      jnp.pad(h2, pad), jnp.pad(h3, pad), off)
    return out
